# baseline (speedup 1.0000x reference)
import functools

import jax
import jax.numpy as jnp
from jax import lax
from jax.experimental import pallas as pl
from jax.experimental.pallas import tpu as pltpu
from jax.experimental.pallas import tpu_sc as plsc

B = 16384
V = 1000000
D = 16
F = 16
L1 = 64
L2 = 32
G8 = 8
VP = 131072
W = D * G8

NC = 2
NS = 16
NW = NC * NS
BPW = B // NW
CH = 256
NSTEP = 4 * (BPW // CH)

TCOL = 1024
NBLK = VP // TCOL
LASTBLK = (V - 1) // TCOL


def _tc_transpose(tabT):
    def body(*refs):
        ins, out_ref = refs[:G8], refs[G8]
        ident = (lax.broadcasted_iota(jnp.int32, (D, D), 0)
                 == lax.broadcasted_iota(jnp.int32, (D, D), 1)
                 ).astype(jnp.float32)
        for g in range(G8):
            xt = lax.dot_general(ins[g][...], ident, (((0,), (0,)), ((), ())),
                                 precision=lax.Precision.HIGHEST,
                                 preferred_element_type=jnp.float32)
            out_ref[:, g * D:(g + 1) * D] = xt

    in_specs = [
        pl.BlockSpec((D, TCOL),
                     functools.partial(
                         lambda g, i: (0, jnp.minimum(g * NBLK + i, LASTBLK)),
                         g))
        for g in range(G8)
    ]
    return pl.pallas_call(
        body,
        grid=(NBLK,),
        in_specs=in_specs,
        out_specs=pl.BlockSpec((TCOL, W), lambda i: (i, 0)),
        out_shape=jax.ShapeDtypeStruct((VP, W), jnp.float32),
    )(*([tabT] * G8))


def _sc_gather(mtab, mfeat, rtab, rfeat, idx_u, idx_v):
    mesh = plsc.VectorSubcoreMesh(core_axis_name="c", subcore_axis_name="s")

    @functools.partial(
        pl.kernel,
        mesh=mesh,
        out_type=[jax.ShapeDtypeStruct((B, W), jnp.float32)] * 4,
        scratch_types=[
            pltpu.VMEM((BPW,), jnp.int32),
            pltpu.VMEM((BPW,), jnp.int32),
            pltpu.VMEM((BPW,), jnp.int32),
            pltpu.VMEM((BPW,), jnp.int32),
            pltpu.VMEM((CH, W), jnp.float32),
            pltpu.VMEM((CH, W), jnp.float32),
            pltpu.SemaphoreType.DMA,
            pltpu.SemaphoreType.DMA,
            pltpu.SemaphoreType.DMA,
        ],
    )
    def k(mtab_h, mfeat_h, rtab_h, rfeat_h, iu_h, iv_h,
          eu_h, fu_h, ev_h, fv_h,
          iu, iv, iu2, iv2, s0, s1, gsem, wsem0, wsem1):
        wid = lax.axis_index("s") * NC + lax.axis_index("c")
        base = wid * BPW
        pltpu.sync_copy(iu_h.at[pl.ds(base, BPW)], iu)
        pltpu.sync_copy(iv_h.at[pl.ds(base, BPW)], iv)
        for j in range(BPW // 16):
            sl = pl.ds(j * 16, 16)
            iu2[sl] = lax.bitwise_and(iu[sl], VP - 1)
            iv2[sl] = lax.bitwise_and(iv[sl], VP - 1)

        tabs = (mtab_h, mfeat_h, rtab_h, rfeat_h)
        idxs = (iu2, iu2, iv2, iv2)
        outs = (eu_h, fu_h, ev_h, fv_h)
        bufs = (s0, s1)
        wsems = (wsem0, wsem1)
        wcps = [None, None]
        for step in range(NSTEP):
            t, c = step // (BPW // CH), step % (BPW // CH)
            p = step % 2
            if wcps[p] is not None:
                wcps[p].wait()
            g = pltpu.async_copy(
                tabs[t].at[idxs[t].at[pl.ds(c * CH, CH)]], bufs[p], gsem)
            g.wait()
            wcps[p] = pltpu.async_copy(
                bufs[p], outs[t].at[pl.ds(base + c * CH, CH)], wsems[p])
        wcps[0].wait()
        wcps[1].wait()

    return k(mtab, mfeat, rtab, rfeat, idx_u, idx_v)


BT = 2048


def _select16(rows, g):
    acc = jnp.zeros((rows.shape[0], D), jnp.float32)
    for kk in range(G8):
        acc = acc + jnp.where(g == kk, rows[:, kk * D:(kk + 1) * D], 0.0)
    return acc


def _tc_body(x_ref, eu_ref, fu_ref, ev_ref, fv_ref,
             mw1, mb1, mw2, mb2, mw3, mb3,
             rw1, rb1, rw2, rb2, rw3, rb3, out_ref):
    gu = lax.shift_right_logical(x_ref[:, 0:1], 17)
    gv = lax.shift_right_logical(x_ref[:, 1:2], 17)
    eu = _select16(eu_ref[...], gu)
    fu = _select16(fu_ref[...], gu)
    ev = _select16(ev_ref[...], gv)
    fv = _select16(fv_ref[...], gv)

    def mlp(f, w1, b1, w2, b2, w3, b3):
        h = jnp.dot(f, w1[...], precision=lax.Precision.HIGHEST,
                    preferred_element_type=jnp.float32) + b1[...]
        h = jnp.maximum(h, 0.0)
        h = jnp.dot(h, w2[...], precision=lax.Precision.HIGHEST,
                    preferred_element_type=jnp.float32) + b2[...]
        h = jnp.maximum(h, 0.0)
        return jnp.dot(h, w3[...], precision=lax.Precision.HIGHEST,
                       preferred_element_type=jnp.float32) + b3[...]

    u = eu + mlp(fu, mw1, mb1, mw2, mb2, mw3, mb3)
    v = ev + mlp(fv, rw1, rb1, rw2, rb2, rw3, rb3)
    out_ref[...] = jnp.sum(u * v, axis=1, keepdims=True)


def _tc_mlp_dot(x, eu, fu, ev, fv,
                m_w1, m_b1, m_w2, m_b2, m_w3, m_b3,
                r_w1, r_b1, r_w2, r_b2, r_w3, r_b3):
    row_spec = pl.BlockSpec((BT, W), lambda i: (i, 0))

    def full(shape):
        return pl.BlockSpec(shape, lambda i: tuple(0 for _ in shape))

    out = pl.pallas_call(
        _tc_body,
        grid=(B // BT,),
        in_specs=[
            pl.BlockSpec((BT, 2), lambda i: (i, 0)),
            row_spec, row_spec, row_spec, row_spec,
            full((F, L1)), full((1, L1)), full((L1, L2)), full((1, L2)),
            full((L2, D)), full((1, D)),
            full((F, L1)), full((1, L1)), full((L1, L2)), full((1, L2)),
            full((L2, D)), full((1, D)),
        ],
        out_specs=pl.BlockSpec((BT, 1), lambda i: (i, 0)),
        out_shape=jax.ShapeDtypeStruct((B, 1), jnp.float32),
    )(x, eu, fu, ev, fv,
      m_w1, m_b1.reshape(1, L1), m_w2, m_b2.reshape(1, L2),
      m_w3, m_b3.reshape(1, D),
      r_w1, r_b1.reshape(1, L1), r_w2, r_b2.reshape(1, L2),
      r_w3, r_b3.reshape(1, D))
    return out.reshape(B)


def kernel(x, module_table, module_feats, m_w1, m_b1, m_w2, m_b2, m_w3, m_b3,
           runtime_table, runtime_feats, r_w1, r_b1, r_w2, r_b2, r_w3, r_b3):
    idx_u = x[:, 0]
    idx_v = x[:, 1]
    mt = _tc_transpose(module_table.T)
    mf = _tc_transpose(module_feats.T)
    rt = _tc_transpose(runtime_table.T)
    rf = _tc_transpose(runtime_feats.T)
    eu, fu, ev, fv = _sc_gather(mt, mf, rt, rf, idx_u, idx_v)
    return _tc_mlp_dot(x, eu, fu, ev, fv,
                       m_w1, m_b1, m_w2, m_b2, m_w3, m_b3,
                       r_w1, r_b1, r_w2, r_b2, r_w3, r_b3)


# speedup vs baseline: 1.4556x; 1.4556x over previous
import functools

import jax
import jax.numpy as jnp
from jax import lax
from jax.experimental import pallas as pl
from jax.experimental.pallas import tpu as pltpu
from jax.experimental.pallas import tpu_sc as plsc

B = 16384
V = 1000000
D = 16
F = 16
L1 = 64
L2 = 32

NC = 2
NS = 16
NW = NC * NS
BPW = B // NW


def _sc_gather(mtab, mfeat, rtab, rfeat, idx_u, idx_v):
    mesh = plsc.VectorSubcoreMesh(core_axis_name="c", subcore_axis_name="s")

    @functools.partial(
        pl.kernel,
        mesh=mesh,
        compiler_params=pltpu.CompilerParams(use_tc_tiling_on_sc=False),
        out_type=[jax.ShapeDtypeStruct((B, D), jnp.float32)] * 4,
        scratch_types=[
            pltpu.VMEM((BPW,), jnp.int32),
            pltpu.VMEM((BPW,), jnp.int32),
            pltpu.VMEM((BPW, D), jnp.float32),
            pltpu.VMEM((BPW, D), jnp.float32),
            pltpu.VMEM((BPW, D), jnp.float32),
            pltpu.VMEM((BPW, D), jnp.float32),
            pltpu.SemaphoreType.DMA,
        ],
    )
    def k(mtab_h, mfeat_h, rtab_h, rfeat_h, iu_h, iv_h,
          eu_h, fu_h, ev_h, fv_h,
          iu, iv, eu, fu, ev, fv, sem):
        wid = lax.axis_index("s") * NC + lax.axis_index("c")
        base = wid * BPW
        pltpu.sync_copy(iu_h.at[pl.ds(base, BPW)], iu)
        pltpu.sync_copy(iv_h.at[pl.ds(base, BPW)], iv)
        c1 = pltpu.async_copy(mtab_h.at[iu], eu, sem)
        c2 = pltpu.async_copy(mfeat_h.at[iu], fu, sem)
        c3 = pltpu.async_copy(rtab_h.at[iv], ev, sem)
        c4 = pltpu.async_copy(rfeat_h.at[iv], fv, sem)
        c1.wait()
        c2.wait()
        c3.wait()
        c4.wait()
        pltpu.sync_copy(eu, eu_h.at[pl.ds(base, BPW)])
        pltpu.sync_copy(fu, fu_h.at[pl.ds(base, BPW)])
        pltpu.sync_copy(ev, ev_h.at[pl.ds(base, BPW)])
        pltpu.sync_copy(fv, fv_h.at[pl.ds(base, BPW)])

    return k(mtab, mfeat, rtab, rfeat, idx_u, idx_v)


BT = 2048


def _tc_body(eu_ref, fu_ref, ev_ref, fv_ref,
             mw1, mb1, mw2, mb2, mw3, mb3,
             rw1, rb1, rw2, rb2, rw3, rb3, out_ref):
    def mlp(f, w1, b1, w2, b2, w3, b3):
        h = jnp.dot(f, w1[...], precision=lax.Precision.HIGHEST,
                    preferred_element_type=jnp.float32) + b1[...]
        h = jnp.maximum(h, 0.0)
        h = jnp.dot(h, w2[...], precision=lax.Precision.HIGHEST,
                    preferred_element_type=jnp.float32) + b2[...]
        h = jnp.maximum(h, 0.0)
        return jnp.dot(h, w3[...], precision=lax.Precision.HIGHEST,
                       preferred_element_type=jnp.float32) + b3[...]

    u = eu_ref[...] + mlp(fu_ref[...], mw1, mb1, mw2, mb2, mw3, mb3)
    v = ev_ref[...] + mlp(fv_ref[...], rw1, rb1, rw2, rb2, rw3, rb3)
    out_ref[...] = jnp.sum(u * v, axis=1, keepdims=True)


def _tc_mlp_dot(eu, fu, ev, fv,
                m_w1, m_b1, m_w2, m_b2, m_w3, m_b3,
                r_w1, r_b1, r_w2, r_b2, r_w3, r_b3):
    row_spec = pl.BlockSpec((BT, D), lambda i: (i, 0))

    def full(shape):
        return pl.BlockSpec(shape, lambda i: tuple(0 for _ in shape))

    out = pl.pallas_call(
        _tc_body,
        grid=(B // BT,),
        in_specs=[
            row_spec, row_spec, row_spec, row_spec,
            full((F, L1)), full((1, L1)), full((L1, L2)), full((1, L2)),
            full((L2, D)), full((1, D)),
            full((F, L1)), full((1, L1)), full((L1, L2)), full((1, L2)),
            full((L2, D)), full((1, D)),
        ],
        out_specs=pl.BlockSpec((BT, 1), lambda i: (i, 0)),
        out_shape=jax.ShapeDtypeStruct((B, 1), jnp.float32),
    )(eu, fu, ev, fv,
      m_w1, m_b1.reshape(1, L1), m_w2, m_b2.reshape(1, L2),
      m_w3, m_b3.reshape(1, D),
      r_w1, r_b1.reshape(1, L1), r_w2, r_b2.reshape(1, L2),
      r_w3, r_b3.reshape(1, D))
    return out.reshape(B)


def kernel(x, module_table, module_feats, m_w1, m_b1, m_w2, m_b2, m_w3, m_b3,
           runtime_table, runtime_feats, r_w1, r_b1, r_w2, r_b2, r_w3, r_b3):
    idx_u = x[:, 0]
    idx_v = x[:, 1]
    eu, fu, ev, fv = _sc_gather(module_table, module_feats,
                                runtime_table, runtime_feats, idx_u, idx_v)
    return _tc_mlp_dot(eu, fu, ev, fv,
                       m_w1, m_b1, m_w2, m_b2, m_w3, m_b3,
                       r_w1, r_b1, r_w2, r_b2, r_w3, r_b3)
